# Initial kernel scaffold; baseline (speedup 1.0000x reference)
import functools

import jax
import jax.numpy as jnp
from jax import lax
from jax.experimental import pallas as pl
from jax.experimental.pallas import tpu as pltpu
from jax.experimental.pallas import tpu_sc as plsc

_VOCAB = 1000000
_D = 64
_B = 16384
_L = 50
_NTOK = _B * _L
_NC = 2
_NS = 16
_NW = _NC * _NS
_B_PER_W = _NTOK // _NW
_CHUNK = 800
_NCHUNK = _B_PER_W // _CHUNK


def _make_kernel():
    mesh = plsc.VectorSubcoreMesh(core_axis_name="c", subcore_axis_name="s")

    @functools.partial(
        pl.kernel,
        mesh=mesh,
        out_type=jax.ShapeDtypeStruct((_NTOK, _D), jnp.float32),
        scratch_types=[
            pltpu.VMEM((_CHUNK,), jnp.int32),
            pltpu.VMEM((_CHUNK, _D), jnp.float32),
            pltpu.SemaphoreType.DMA,
        ],
    )
    def emb(idx_hbm, table_hbm, out_hbm, idx_v, rows_v, sem):
        wid = lax.axis_index("s") * _NC + lax.axis_index("c")
        base = wid * _B_PER_W

        def body(g, carry):
            off = base + g * _CHUNK
            pltpu.sync_copy(idx_hbm.at[pl.ds(off, _CHUNK)], idx_v)
            pltpu.async_copy(table_hbm.at[idx_v], rows_v, sem).wait()
            pltpu.sync_copy(rows_v, out_hbm.at[pl.ds(off, _CHUNK)])
            return carry

        lax.fori_loop(0, _NCHUNK, body, 0)

    return emb


_emb = _make_kernel()


@jax.jit
def kernel(x, table):
    idx = x.reshape(_NTOK)
    out = _emb(idx, table)
    return out.reshape(_B, _L, _D)


# speedup vs baseline: 1.8308x; 1.8308x over previous
import functools

import jax
import jax.numpy as jnp
from jax import lax
from jax.experimental import pallas as pl
from jax.experimental.pallas import tpu as pltpu
from jax.experimental.pallas import tpu_sc as plsc

_VOCAB = 1000000
_D = 64
_B = 16384
_L = 50
_NTOK = _B * _L
_NC = 2
_NS = 16
_NW = _NC * _NS
_B_PER_W = _NTOK // _NW
_CHUNK = 800
_NCHUNK = _B_PER_W // _CHUNK


def _make_kernel():
    mesh = plsc.VectorSubcoreMesh(core_axis_name="c", subcore_axis_name="s")

    @functools.partial(
        pl.kernel,
        mesh=mesh,
        out_type=jax.ShapeDtypeStruct((_NTOK, _D), jnp.float32),
        scratch_types=[
            pltpu.VMEM((_CHUNK,), jnp.int32),
            pltpu.VMEM((_CHUNK, _D), jnp.float32),
            pltpu.SemaphoreType.DMA,
        ],
        compiler_params=pltpu.CompilerParams(use_tc_tiling_on_sc=False),
    )
    def emb(idx_hbm, table_hbm, out_hbm, idx_v, rows_v, sem):
        wid = lax.axis_index("s") * _NC + lax.axis_index("c")
        base = wid * _B_PER_W

        def body(g, carry):
            off = base + g * _CHUNK
            pltpu.sync_copy(idx_hbm.at[pl.ds(off, _CHUNK)], idx_v)
            pltpu.async_copy(table_hbm.at[idx_v], rows_v, sem).wait()
            pltpu.sync_copy(rows_v, out_hbm.at[pl.ds(off, _CHUNK)])
            return carry

        lax.fori_loop(0, _NCHUNK, body, 0)

    return emb


_emb = _make_kernel()


@jax.jit
def kernel(x, table):
    idx = x.reshape(_NTOK)
    out = _emb(idx, table)
    return out.reshape(_B, _L, _D)


# speedup vs baseline: 1.8611x; 1.0165x over previous
import functools

import jax
import jax.numpy as jnp
from jax import lax
from jax.experimental import pallas as pl
from jax.experimental.pallas import tpu as pltpu
from jax.experimental.pallas import tpu_sc as plsc

_VOCAB = 1000000
_D = 64
_B = 16384
_L = 50
_NTOK = _B * _L
_NC = 2
_NS = 16
_NW = _NC * _NS
_B_PER_W = _NTOK // _NW
_CHUNK = 800
_NB = 2
_NCHUNK = _B_PER_W // _CHUNK
_NOUTER = _NCHUNK // _NB


def _make_kernel():
    mesh = plsc.VectorSubcoreMesh(core_axis_name="c", subcore_axis_name="s")

    @functools.partial(
        pl.kernel,
        mesh=mesh,
        out_type=jax.ShapeDtypeStruct((_NTOK, _D), jnp.float32),
        scratch_types=(
            [pltpu.VMEM((_CHUNK,), jnp.int32)] * _NB
            + [pltpu.VMEM((_CHUNK, _D), jnp.float32)] * _NB
            + [pltpu.SemaphoreType.DMA] * (2 * _NB)
        ),
        compiler_params=pltpu.CompilerParams(use_tc_tiling_on_sc=False),
    )
    def emb(idx_hbm, table_hbm, out_hbm, *scratch):
        idx_vs = scratch[:_NB]
        rows_vs = scratch[_NB:2 * _NB]
        gsems = scratch[2 * _NB:3 * _NB]
        wsems = scratch[3 * _NB:]
        wid = lax.axis_index("s") * _NC + lax.axis_index("c")
        base = wid * _B_PER_W

        def gather_start(c, b):
            off = base + c * _CHUNK
            pltpu.sync_copy(idx_hbm.at[pl.ds(off, _CHUNK)], idx_vs[b])
            pltpu.async_copy(table_hbm.at[idx_vs[b]], rows_vs[b], gsems[b])

        def gather_wait(b):
            pltpu.make_async_copy(table_hbm.at[idx_vs[b]], rows_vs[b],
                                  gsems[b]).wait()

        def write_start(c, b):
            off = base + c * _CHUNK
            pltpu.async_copy(rows_vs[b], out_hbm.at[pl.ds(off, _CHUNK)],
                             wsems[b])

        def write_wait(c, b):
            off = base + c * _CHUNK
            pltpu.make_async_copy(rows_vs[b],
                                  out_hbm.at[pl.ds(off, _CHUNK)],
                                  wsems[b]).wait()

        def body(t, carry):
            for b in range(_NB):
                c = t * _NB + b
                bp = (b - 1) % _NB
                pl.when(c >= _NB)(lambda: write_wait(c - _NB, b))
                gather_start(c, b)
                def drain_prev():
                    gather_wait(bp)
                    write_start(c - 1, bp)
                pl.when(c >= 1)(drain_prev)
            return carry

        lax.fori_loop(0, _NOUTER, body, 0)

        b_last = (_NCHUNK - 1) % _NB
        gather_wait(b_last)
        write_start(_NCHUNK - 1, b_last)
        for b in range(_NB):
            c = _NCHUNK - _NB + b
            write_wait(c, b)

    return emb


_emb = _make_kernel()


@jax.jit
def kernel(x, table):
    idx = x.reshape(_NTOK)
    out = _emb(idx, table)
    return out.reshape(_B, _L, _D)


# speedup vs baseline: 1.8743x; 1.0071x over previous
import functools

import jax
import jax.numpy as jnp
from jax import lax
from jax.experimental import pallas as pl
from jax.experimental.pallas import tpu as pltpu
from jax.experimental.pallas import tpu_sc as plsc

_VOCAB = 1000000
_D = 64
_B = 16384
_L = 50
_NTOK = _B * _L
_NC = 2
_NS = 16
_NW = _NC * _NS
_B_PER_W = _NTOK // _NW
_CHUNK = 400
_NB = 4
_NCHUNK = _B_PER_W // _CHUNK
_NOUTER = _NCHUNK // _NB


def _make_kernel():
    mesh = plsc.VectorSubcoreMesh(core_axis_name="c", subcore_axis_name="s")

    @functools.partial(
        pl.kernel,
        mesh=mesh,
        out_type=jax.ShapeDtypeStruct((_NTOK, _D), jnp.float32),
        scratch_types=(
            [pltpu.VMEM((_B_PER_W,), jnp.int32)]
            + [pltpu.VMEM((_CHUNK, _D), jnp.float32)] * _NB
            + [pltpu.SemaphoreType.DMA] * (2 * _NB)
        ),
        compiler_params=pltpu.CompilerParams(use_tc_tiling_on_sc=False),
    )
    def emb(idx_hbm, table_hbm, out_hbm, *scratch):
        idx_all = scratch[0]
        rows_vs = scratch[1:1 + _NB]
        gsems = scratch[1 + _NB:1 + 2 * _NB]
        wsems = scratch[1 + 2 * _NB:]
        wid = lax.axis_index("s") * _NC + lax.axis_index("c")
        base = wid * _B_PER_W

        pltpu.sync_copy(idx_hbm.at[pl.ds(base, _B_PER_W)], idx_all)

        def gather_start(c, b):
            pltpu.async_copy(
                table_hbm.at[idx_all.at[pl.ds(c * _CHUNK, _CHUNK)]],
                rows_vs[b], gsems[b])

        def gather_wait(c, b):
            pltpu.make_async_copy(
                table_hbm.at[idx_all.at[pl.ds(c * _CHUNK, _CHUNK)]],
                rows_vs[b], gsems[b]).wait()

        def write_start(c, b):
            pltpu.async_copy(rows_vs[b],
                             out_hbm.at[pl.ds(base + c * _CHUNK, _CHUNK)],
                             wsems[b])

        def write_wait(c, b):
            pltpu.make_async_copy(rows_vs[b],
                                  out_hbm.at[pl.ds(base + c * _CHUNK, _CHUNK)],
                                  wsems[b]).wait()

        def body(t, carry):
            for b in range(_NB):
                c = t * _NB + b
                pl.when(c >= _NB)(lambda: write_wait(c - _NB, b))
                gather_start(c, b)
                bq = (b + 1) % _NB
                def drain():
                    cq = c - (_NB - 1)
                    gather_wait(cq, bq)
                    write_start(cq, bq)
                pl.when(c >= _NB - 1)(drain)
            return carry

        lax.fori_loop(0, _NOUTER, body, 0)

        for k in range(_NB - 1):
            c = _NCHUNK - (_NB - 1) + k
            b = c % _NB
            gather_wait(c, b)
            write_start(c, b)
        for k in range(_NB):
            c = _NCHUNK - _NB + k
            write_wait(c, c % _NB)

    return emb


_emb = _make_kernel()


@jax.jit
def kernel(x, table):
    idx = x.reshape(_NTOK)
    out = _emb(idx, table)
    return out.reshape(_B, _L, _D)


# speedup vs baseline: 1.8745x; 1.0001x over previous
import functools

import jax
import jax.numpy as jnp
from jax import lax
from jax.experimental import pallas as pl
from jax.experimental.pallas import tpu as pltpu
from jax.experimental.pallas import tpu_sc as plsc

_VOCAB = 1000000
_D = 64
_B = 16384
_L = 50
_NTOK = _B * _L
_NC = 2
_NS = 16
_NW = _NC * _NS
_B_PER_W = _NTOK // _NW
_CHUNK = 800
_NB = 2
_NCHUNK = _B_PER_W // _CHUNK
_NOUTER = _NCHUNK // _NB


def _make_kernel():
    mesh = plsc.VectorSubcoreMesh(core_axis_name="c", subcore_axis_name="s")

    @functools.partial(
        pl.kernel,
        mesh=mesh,
        out_type=jax.ShapeDtypeStruct((_NTOK, _D), jnp.float32),
        scratch_types=(
            [pltpu.VMEM((_B_PER_W,), jnp.int32)]
            + [pltpu.VMEM((_CHUNK, _D), jnp.float32)] * _NB
            + [pltpu.SemaphoreType.DMA] * (2 * _NB)
        ),
        compiler_params=pltpu.CompilerParams(use_tc_tiling_on_sc=False),
    )
    def emb(idx_hbm, table_hbm, out_hbm, *scratch):
        idx_all = scratch[0]
        rows_vs = scratch[1:1 + _NB]
        gsems = scratch[1 + _NB:1 + 2 * _NB]
        wsems = scratch[1 + 2 * _NB:]
        wid = lax.axis_index("s") * _NC + lax.axis_index("c")
        base = wid * _B_PER_W

        pltpu.sync_copy(idx_hbm.at[pl.ds(base, _B_PER_W)], idx_all)

        def gather_start(c, b):
            pltpu.async_copy(
                table_hbm.at[idx_all.at[pl.ds(c * _CHUNK, _CHUNK)]],
                rows_vs[b], gsems[b])

        def gather_wait(c, b):
            pltpu.make_async_copy(
                table_hbm.at[idx_all.at[pl.ds(c * _CHUNK, _CHUNK)]],
                rows_vs[b], gsems[b]).wait()

        def write_start(c, b):
            pltpu.async_copy(rows_vs[b],
                             out_hbm.at[pl.ds(base + c * _CHUNK, _CHUNK)],
                             wsems[b])

        def write_wait(c, b):
            pltpu.make_async_copy(rows_vs[b],
                                  out_hbm.at[pl.ds(base + c * _CHUNK, _CHUNK)],
                                  wsems[b]).wait()

        def body(t, carry):
            for b in range(_NB):
                c = t * _NB + b
                pl.when(c >= _NB)(lambda: write_wait(c - _NB, b))
                gather_start(c, b)
                bq = (b + 1) % _NB
                def drain():
                    cq = c - (_NB - 1)
                    gather_wait(cq, bq)
                    write_start(cq, bq)
                pl.when(c >= _NB - 1)(drain)
            return carry

        lax.fori_loop(0, _NOUTER, body, 0)

        for k in range(_NB - 1):
            c = _NCHUNK - (_NB - 1) + k
            b = c % _NB
            gather_wait(c, b)
            write_start(c, b)
        for k in range(_NB):
            c = _NCHUNK - _NB + k
            write_wait(c, c % _NB)

    return emb


_emb = _make_kernel()


@jax.jit
def kernel(x, table):
    idx = x.reshape(_NTOK)
    out = _emb(idx, table)
    return out.reshape(_B, _L, _D)
